# baseline (speedup 1.0000x reference)
import jax
import jax.numpy as jnp
import numpy as np
from jax import lax
from jax.experimental import pallas as pl
from jax.experimental.pallas import tpu as pltpu
from jax.experimental.pallas import tpu_sc as plsc

N = 10000
D = 128
E = 320000
HW = D // 2

NC = 2
NS = 16
NW = NC * NS
CH = 128
K = 80
PCH = 64
PK = 160
KP = 40
PHF = PK // KP
TOT_CHUNKS = NW * PK
E_PAD = TOT_CHUNKS * PCH
NBUF = 4

ACC_ROWS = 10240
ZR = ACC_ROWS // NS

_MESH = plsc.VectorSubcoreMesh(core_axis_name="c", subcore_axis_name="s")

_PERM_NP = np.zeros((D, D), np.float32)
for _k in range(D // 32):
    for _i in range(16):
        _PERM_NP[32 * _k + _i, 32 * _k + 2 * _i] = 1.0
        _PERM_NP[32 * _k + 16 + _i, 32 * _k + 2 * _i + 1] = 1.0


def _deg_body(colt_hbm, zeros_hbm, ones_hbm, out_hbm, coli, ones_v, acc):
    c = lax.axis_index("c")
    s = lax.axis_index("s")
    wid = c * NS + s
    pltpu.sync_copy(zeros_hbm, acc.at[pl.ds(s * ZR, ZR)])
    pltpu.sync_copy(colt_hbm.at[wid], coli)
    pltpu.sync_copy(ones_hbm, ones_v)
    plsc.subcore_barrier()

    @pl.loop(0, K)
    def _(j):
        pltpu.sync_copy(ones_v, acc.at[coli.at[j]], add=True)

    plsc.subcore_barrier()
    pltpu.sync_copy(acc.at[pl.ds(s * ZR, ZR)], out_hbm.at[c, pl.ds(s * ZR, ZR)])


_sc_deg = pl.kernel(
    _deg_body,
    out_type=jax.ShapeDtypeStruct((NC, ACC_ROWS, D), jnp.float32),
    mesh=_MESH,
    scratch_types=[
        pltpu.VMEM((K, CH), jnp.int32),
        pltpu.VMEM((CH, D), jnp.float32),
        pltpu.VMEM_SHARED((ACC_ROWS, D), jnp.float32),
    ],
)


def _prop_body(h_hbm, rowf_hbm, colf_hbm, zeros_hbm, out_hbm,
               rowi, coli, rowsw, stag, acc, semg):
    c = lax.axis_index("c")
    s = lax.axis_index("s")
    pltpu.sync_copy(zeros_hbm, acc.at[pl.ds(s * ZR, ZR)])
    plsc.subcore_barrier()

    wid = c * NS + s
    base = wid * PK

    for ph in range(PHF):
        pltpu.sync_copy(rowf_hbm.at[pl.ds(base + ph * KP, KP)], rowi)
        pltpu.sync_copy(colf_hbm.at[pl.ds(base + ph * KP, KP)], coli)

        for j in range(NBUF - 1):
            pltpu.async_copy(h_hbm.at[rowi.at[j]], rowsw.at[j], semg)

        @pl.loop(0, KP)
        def _(j):
            p = lax.rem(j, NBUF)
            pltpu.make_async_copy(
                h_hbm.at[rowi.at[j]], rowsw.at[p], semg).wait()

            @pl.when(j + NBUF - 1 < KP)
            def _():
                pltpu.async_copy(h_hbm.at[rowi.at[j + NBUF - 1]],
                                 rowsw.at[lax.rem(j + NBUF - 1, NBUF)],
                                 semg)

            @pl.loop(0, PCH, unroll=8)
            def _(r):
                for k in range(D // 32):
                    w = rowsw[p, r, pl.ds(16 * k, 16)]
                    a = plsc.bitcast(lax.shift_left(w, 16), jnp.float32)
                    b = plsc.bitcast(
                        lax.bitwise_and(w, jnp.int32(-65536)), jnp.float32)
                    stag[r, pl.ds(32 * k, 16)] = a
                    stag[r, pl.ds(32 * k + 16, 16)] = b

            pltpu.sync_copy(stag, acc.at[coli.at[j]], add=True)

    plsc.subcore_barrier()
    pltpu.sync_copy(acc.at[pl.ds(s * ZR, ZR)], out_hbm.at[c, pl.ds(s * ZR, ZR)])


_sc_prop = pl.kernel(
    _prop_body,
    out_type=jax.ShapeDtypeStruct((NC, ACC_ROWS, D), jnp.float32),
    mesh=_MESH,
    scratch_types=[
        pltpu.VMEM((KP, PCH), jnp.int32),
        pltpu.VMEM((KP, PCH), jnp.int32),
        pltpu.VMEM((NBUF, PCH, HW), jnp.int32),
        pltpu.VMEM((PCH, D), jnp.float32),
        pltpu.VMEM_SHARED((ACC_ROWS, D), jnp.float32),
        pltpu.SemaphoreType.DMA,
    ],
    compiler_params=pltpu.CompilerParams(use_tc_tiling_on_sc=False,
                                         needs_layout_passes=False),
)

_BLK = 2000
_GRID = N // _BLK
_EPS = 1e-12


def _tc_a_body(x_ref, w_ref, b_ref, degp_ref, perm_ref, h1_ref, h1p_ref,
               dinv_ref):
    xb = x_ref[...]
    nrm = jnp.sqrt(jnp.sum(xb * xb, axis=1, keepdims=True))
    xn = xb / jnp.maximum(nrm, _EPS)
    z = jnp.dot(xn, w_ref[...], preferred_element_type=jnp.float32) + b_ref[...]
    deg = degp_ref[0] + degp_ref[1] + 1.0
    dinv = lax.rsqrt(deg)
    dinv_ref[...] = dinv[:, 0:16]
    h1 = z * dinv
    h1_ref[...] = h1
    h1p_ref[...] = jnp.dot(h1, perm_ref[...],
                           preferred_element_type=jnp.float32).astype(jnp.bfloat16)


_tc_a = pl.pallas_call(
    _tc_a_body,
    grid=(_GRID,),
    in_specs=[
        pl.BlockSpec((_BLK, D), lambda i: (i, 0)),
        pl.BlockSpec((D, D), lambda i: (0, 0)),
        pl.BlockSpec((1, D), lambda i: (0, 0)),
        pl.BlockSpec((NC, _BLK, D), lambda i: (0, i, 0)),
        pl.BlockSpec((D, D), lambda i: (0, 0)),
    ],
    out_specs=[
        pl.BlockSpec((_BLK, D), lambda i: (i, 0)),
        pl.BlockSpec((_BLK, D), lambda i: (i, 0)),
        pl.BlockSpec((_BLK, 16), lambda i: (i, 0)),
    ],
    out_shape=[
        jax.ShapeDtypeStruct((N, D), jnp.float32),
        jax.ShapeDtypeStruct((N, D), jnp.bfloat16),
        jax.ShapeDtypeStruct((N, 16), jnp.float32),
    ],
)


def _tc_b_body(aggp_ref, h1_ref, dinv_ref, wmu_ref, bmu_ref, wvar_ref,
               bvar_ref, perm_ref, hmu_ref, hvar_ref, hmup_ref, hvarp_ref):
    dinv = dinv_ref[...][:, 0:1]
    agg = aggp_ref[0] + aggp_ref[1] + h1_ref[...]
    hid = jnp.maximum(agg * dinv, 0.0)
    perm = perm_ref[...]
    zmu = jnp.dot(hid, wmu_ref[...], preferred_element_type=jnp.float32) + bmu_ref[...]
    nmu = jnp.sqrt(jnp.sum(zmu * zmu, axis=1, keepdims=True))
    hmu = zmu / jnp.maximum(nmu, _EPS) * dinv
    hmu_ref[...] = hmu
    hmup_ref[...] = jnp.dot(hmu, perm,
                            preferred_element_type=jnp.float32).astype(jnp.bfloat16)
    zva = jnp.dot(hid, wvar_ref[...], preferred_element_type=jnp.float32) + bvar_ref[...]
    nva = jnp.sqrt(jnp.sum(zva * zva, axis=1, keepdims=True))
    hvar = zva / jnp.maximum(nva, _EPS) * dinv
    hvar_ref[...] = hvar
    hvarp_ref[...] = jnp.dot(hvar, perm,
                             preferred_element_type=jnp.float32).astype(jnp.bfloat16)


_tc_b = pl.pallas_call(
    _tc_b_body,
    grid=(_GRID,),
    in_specs=[
        pl.BlockSpec((NC, _BLK, D), lambda i: (0, i, 0)),
        pl.BlockSpec((_BLK, D), lambda i: (i, 0)),
        pl.BlockSpec((_BLK, 16), lambda i: (i, 0)),
        pl.BlockSpec((D, D), lambda i: (0, 0)),
        pl.BlockSpec((1, D), lambda i: (0, 0)),
        pl.BlockSpec((D, D), lambda i: (0, 0)),
        pl.BlockSpec((1, D), lambda i: (0, 0)),
        pl.BlockSpec((D, D), lambda i: (0, 0)),
    ],
    out_specs=[
        pl.BlockSpec((_BLK, D), lambda i: (i, 0)),
        pl.BlockSpec((_BLK, D), lambda i: (i, 0)),
        pl.BlockSpec((_BLK, D), lambda i: (i, 0)),
        pl.BlockSpec((_BLK, D), lambda i: (i, 0)),
    ],
    out_shape=[
        jax.ShapeDtypeStruct((N, D), jnp.float32),
        jax.ShapeDtypeStruct((N, D), jnp.float32),
        jax.ShapeDtypeStruct((N, D), jnp.bfloat16),
        jax.ShapeDtypeStruct((N, D), jnp.bfloat16),
    ],
)


def _tc_c_body(amup_ref, avap_ref, hmu_ref, hvar_ref, dinv_ref,
               mu_ref, var_ref):
    dinv = dinv_ref[...][:, 0:1]
    mu_ref[...] = (amup_ref[0] + amup_ref[1] + hmu_ref[...]) * dinv
    var_ref[...] = (avap_ref[0] + avap_ref[1] + hvar_ref[...]) * dinv


_tc_c = pl.pallas_call(
    _tc_c_body,
    grid=(_GRID,),
    in_specs=[
        pl.BlockSpec((NC, _BLK, D), lambda i: (0, i, 0)),
        pl.BlockSpec((NC, _BLK, D), lambda i: (0, i, 0)),
        pl.BlockSpec((_BLK, D), lambda i: (i, 0)),
        pl.BlockSpec((_BLK, D), lambda i: (i, 0)),
        pl.BlockSpec((_BLK, 16), lambda i: (i, 0)),
    ],
    out_specs=[
        pl.BlockSpec((_BLK, D), lambda i: (i, 0)),
        pl.BlockSpec((_BLK, D), lambda i: (i, 0)),
    ],
    out_shape=[
        jax.ShapeDtypeStruct((N, D), jnp.float32),
        jax.ShapeDtypeStruct((N, D), jnp.float32),
    ],
)


def _pack_words(hp):
    return lax.bitcast_convert_type(hp.reshape(N, HW, 2), jnp.int32)


def kernel(x, edge_index, W0, b0, W_mu, b_mu, W_var, b_var):
    ei = edge_index.astype(jnp.int32)
    pad = E_PAD - E
    rowf = jnp.concatenate([ei[0], jnp.zeros((pad,), jnp.int32)]).reshape(
        TOT_CHUNKS, PCH)
    colf = jnp.concatenate([ei[1], jnp.full((pad,), N, jnp.int32)]).reshape(
        TOT_CHUNKS, PCH)
    colt = colf.reshape(NW, K, CH)

    zerosD = jnp.zeros((ZR, D), jnp.float32)
    onesD = jnp.ones((CH, D), jnp.float32)
    perm = jnp.asarray(_PERM_NP)

    degp = _sc_deg(colt, zerosD, onesD)
    h1, h1p, dinv = _tc_a(x, W0, b0.reshape(1, D), degp, perm)
    aggp = _sc_prop(_pack_words(h1p), rowf, colf, zerosD)
    hmu, hvar, hmup, hvarp = _tc_b(aggp, h1, dinv, W_mu, b_mu.reshape(1, D),
                                   W_var, b_var.reshape(1, D), perm)
    amup = _sc_prop(_pack_words(hmup), rowf, colf, zerosD)
    avap = _sc_prop(_pack_words(hvarp), rowf, colf, zerosD)
    mu, var = _tc_c(amup, avap, hmu, hvar, dinv)
    return (mu, var)


# speedup vs baseline: 1.0824x; 1.0824x over previous
import jax
import jax.numpy as jnp
import numpy as np
from jax import lax
from jax.experimental import pallas as pl
from jax.experimental.pallas import tpu as pltpu
from jax.experimental.pallas import tpu_sc as plsc

N = 10000
D = 128
E = 320000
HW = D // 2

NC = 2
NS = 16
NW = NC * NS
CH = 128
K = 80
PCH = 64
PK = 160
KP = 40
PHF = PK // KP
TOT_CHUNKS = NW * PK
E_PAD = TOT_CHUNKS * PCH
NBUF = 4

ACC_ROWS = 10240
ZR = ACC_ROWS // NS

_MESH = plsc.VectorSubcoreMesh(core_axis_name="c", subcore_axis_name="s")

_PERM_NP = np.zeros((D, D), np.float32)
for _k in range(D // 32):
    for _i in range(16):
        _PERM_NP[32 * _k + _i, 32 * _k + 2 * _i] = 1.0
        _PERM_NP[32 * _k + 16 + _i, 32 * _k + 2 * _i + 1] = 1.0


def _deg_body(colt_hbm, zeros_hbm, ones_hbm, out_hbm, coli, ones_v, acc):
    c = lax.axis_index("c")
    s = lax.axis_index("s")
    wid = c * NS + s
    pltpu.sync_copy(zeros_hbm, acc.at[pl.ds(s * ZR, ZR)])
    pltpu.sync_copy(colt_hbm.at[wid], coli)
    pltpu.sync_copy(ones_hbm, ones_v)
    plsc.subcore_barrier()

    @pl.loop(0, K)
    def _(j):
        pltpu.sync_copy(ones_v, acc.at[coli.at[j]], add=True)

    plsc.subcore_barrier()
    pltpu.sync_copy(acc.at[pl.ds(s * ZR, ZR)], out_hbm.at[c, pl.ds(s * ZR, ZR)])


_sc_deg = pl.kernel(
    _deg_body,
    out_type=jax.ShapeDtypeStruct((NC, ACC_ROWS, D), jnp.float32),
    mesh=_MESH,
    scratch_types=[
        pltpu.VMEM((K, CH), jnp.int32),
        pltpu.VMEM((CH, D), jnp.float32),
        pltpu.VMEM_SHARED((ACC_ROWS, D), jnp.float32),
    ],
)


def _prop_body(h_hbm, rowf_hbm, colf_hbm, zeros_hbm, out_hbm,
               rowi, coli, rowsw, stag, acc, semg, sems):
    c = lax.axis_index("c")
    s = lax.axis_index("s")
    pltpu.sync_copy(zeros_hbm, acc.at[pl.ds(s * ZR, ZR)])
    plsc.subcore_barrier()

    wid = c * NS + s
    base = wid * PK

    for ph in range(PHF):
        pltpu.sync_copy(rowf_hbm.at[pl.ds(base + ph * KP, KP)], rowi)
        pltpu.sync_copy(colf_hbm.at[pl.ds(base + ph * KP, KP)], coli)

        for j in range(NBUF - 1):
            pltpu.async_copy(h_hbm.at[rowi.at[j]], rowsw.at[j], semg)

        @pl.loop(0, KP)
        def _(j):
            p = lax.rem(j, NBUF)
            pltpu.make_async_copy(
                h_hbm.at[rowi.at[j]], rowsw.at[p], semg).wait()

            @pl.when(j + NBUF - 1 < KP)
            def _():
                pltpu.async_copy(h_hbm.at[rowi.at[j + NBUF - 1]],
                                 rowsw.at[lax.rem(j + NBUF - 1, NBUF)],
                                 semg)

            q = lax.rem(j, 2)

            @pl.when(j >= 2)
            def _():
                pltpu.make_async_copy(
                    stag.at[q], acc.at[coli.at[j - 2]], sems).wait()

            @pl.loop(0, PCH, unroll=8)
            def _(r):
                for k in range(D // 32):
                    w = rowsw[p, r, pl.ds(16 * k, 16)]
                    a = plsc.bitcast(lax.shift_left(w, 16), jnp.float32)
                    b = plsc.bitcast(
                        lax.bitwise_and(w, jnp.int32(-65536)), jnp.float32)
                    stag[q, r, pl.ds(32 * k, 16)] = a
                    stag[q, r, pl.ds(32 * k + 16, 16)] = b

            pltpu.async_copy(stag.at[q], acc.at[coli.at[j]], sems, add=True)

        for t in (KP - 2, KP - 1):
            pltpu.make_async_copy(
                stag.at[t % 2], acc.at[coli.at[t]], sems).wait()

    plsc.subcore_barrier()
    pltpu.sync_copy(acc.at[pl.ds(s * ZR, ZR)], out_hbm.at[c, pl.ds(s * ZR, ZR)])


_sc_prop = pl.kernel(
    _prop_body,
    out_type=jax.ShapeDtypeStruct((NC, ACC_ROWS, D), jnp.float32),
    mesh=_MESH,
    scratch_types=[
        pltpu.VMEM((KP, PCH), jnp.int32),
        pltpu.VMEM((KP, PCH), jnp.int32),
        pltpu.VMEM((NBUF, PCH, HW), jnp.int32),
        pltpu.VMEM((2, PCH, D), jnp.float32),
        pltpu.VMEM_SHARED((ACC_ROWS, D), jnp.float32),
        pltpu.SemaphoreType.DMA,
        pltpu.SemaphoreType.DMA,
    ],
    compiler_params=pltpu.CompilerParams(use_tc_tiling_on_sc=False,
                                         needs_layout_passes=False),
)

_BLK = 2000
_GRID = N // _BLK
_EPS = 1e-12


def _tc_a_body(x_ref, w_ref, b_ref, degp_ref, perm_ref, h1_ref, h1p_ref,
               dinv_ref):
    xb = x_ref[...]
    nrm = jnp.sqrt(jnp.sum(xb * xb, axis=1, keepdims=True))
    xn = xb / jnp.maximum(nrm, _EPS)
    z = jnp.dot(xn, w_ref[...], preferred_element_type=jnp.float32) + b_ref[...]
    deg = degp_ref[0] + degp_ref[1] + 1.0
    dinv = lax.rsqrt(deg)
    dinv_ref[...] = dinv[:, 0:16]
    h1 = z * dinv
    h1_ref[...] = h1
    h1p_ref[...] = jnp.dot(h1, perm_ref[...],
                           preferred_element_type=jnp.float32).astype(jnp.bfloat16)


_tc_a = pl.pallas_call(
    _tc_a_body,
    grid=(_GRID,),
    in_specs=[
        pl.BlockSpec((_BLK, D), lambda i: (i, 0)),
        pl.BlockSpec((D, D), lambda i: (0, 0)),
        pl.BlockSpec((1, D), lambda i: (0, 0)),
        pl.BlockSpec((NC, _BLK, D), lambda i: (0, i, 0)),
        pl.BlockSpec((D, D), lambda i: (0, 0)),
    ],
    out_specs=[
        pl.BlockSpec((_BLK, D), lambda i: (i, 0)),
        pl.BlockSpec((_BLK, D), lambda i: (i, 0)),
        pl.BlockSpec((_BLK, 16), lambda i: (i, 0)),
    ],
    out_shape=[
        jax.ShapeDtypeStruct((N, D), jnp.float32),
        jax.ShapeDtypeStruct((N, D), jnp.bfloat16),
        jax.ShapeDtypeStruct((N, 16), jnp.float32),
    ],
)


def _tc_b_body(aggp_ref, h1_ref, dinv_ref, wmu_ref, bmu_ref, wvar_ref,
               bvar_ref, perm_ref, hmu_ref, hvar_ref, hmup_ref, hvarp_ref):
    dinv = dinv_ref[...][:, 0:1]
    agg = aggp_ref[0] + aggp_ref[1] + h1_ref[...]
    hid = jnp.maximum(agg * dinv, 0.0)
    perm = perm_ref[...]
    zmu = jnp.dot(hid, wmu_ref[...], preferred_element_type=jnp.float32) + bmu_ref[...]
    nmu = jnp.sqrt(jnp.sum(zmu * zmu, axis=1, keepdims=True))
    hmu = zmu / jnp.maximum(nmu, _EPS) * dinv
    hmu_ref[...] = hmu
    hmup_ref[...] = jnp.dot(hmu, perm,
                            preferred_element_type=jnp.float32).astype(jnp.bfloat16)
    zva = jnp.dot(hid, wvar_ref[...], preferred_element_type=jnp.float32) + bvar_ref[...]
    nva = jnp.sqrt(jnp.sum(zva * zva, axis=1, keepdims=True))
    hvar = zva / jnp.maximum(nva, _EPS) * dinv
    hvar_ref[...] = hvar
    hvarp_ref[...] = jnp.dot(hvar, perm,
                             preferred_element_type=jnp.float32).astype(jnp.bfloat16)


_tc_b = pl.pallas_call(
    _tc_b_body,
    grid=(_GRID,),
    in_specs=[
        pl.BlockSpec((NC, _BLK, D), lambda i: (0, i, 0)),
        pl.BlockSpec((_BLK, D), lambda i: (i, 0)),
        pl.BlockSpec((_BLK, 16), lambda i: (i, 0)),
        pl.BlockSpec((D, D), lambda i: (0, 0)),
        pl.BlockSpec((1, D), lambda i: (0, 0)),
        pl.BlockSpec((D, D), lambda i: (0, 0)),
        pl.BlockSpec((1, D), lambda i: (0, 0)),
        pl.BlockSpec((D, D), lambda i: (0, 0)),
    ],
    out_specs=[
        pl.BlockSpec((_BLK, D), lambda i: (i, 0)),
        pl.BlockSpec((_BLK, D), lambda i: (i, 0)),
        pl.BlockSpec((_BLK, D), lambda i: (i, 0)),
        pl.BlockSpec((_BLK, D), lambda i: (i, 0)),
    ],
    out_shape=[
        jax.ShapeDtypeStruct((N, D), jnp.float32),
        jax.ShapeDtypeStruct((N, D), jnp.float32),
        jax.ShapeDtypeStruct((N, D), jnp.bfloat16),
        jax.ShapeDtypeStruct((N, D), jnp.bfloat16),
    ],
)


def _tc_c_body(amup_ref, avap_ref, hmu_ref, hvar_ref, dinv_ref,
               mu_ref, var_ref):
    dinv = dinv_ref[...][:, 0:1]
    mu_ref[...] = (amup_ref[0] + amup_ref[1] + hmu_ref[...]) * dinv
    var_ref[...] = (avap_ref[0] + avap_ref[1] + hvar_ref[...]) * dinv


_tc_c = pl.pallas_call(
    _tc_c_body,
    grid=(_GRID,),
    in_specs=[
        pl.BlockSpec((NC, _BLK, D), lambda i: (0, i, 0)),
        pl.BlockSpec((NC, _BLK, D), lambda i: (0, i, 0)),
        pl.BlockSpec((_BLK, D), lambda i: (i, 0)),
        pl.BlockSpec((_BLK, D), lambda i: (i, 0)),
        pl.BlockSpec((_BLK, 16), lambda i: (i, 0)),
    ],
    out_specs=[
        pl.BlockSpec((_BLK, D), lambda i: (i, 0)),
        pl.BlockSpec((_BLK, D), lambda i: (i, 0)),
    ],
    out_shape=[
        jax.ShapeDtypeStruct((N, D), jnp.float32),
        jax.ShapeDtypeStruct((N, D), jnp.float32),
    ],
)


def _pack_words(hp):
    return lax.bitcast_convert_type(hp.reshape(N, HW, 2), jnp.int32)


def kernel(x, edge_index, W0, b0, W_mu, b_mu, W_var, b_var):
    ei = edge_index.astype(jnp.int32)
    pad = E_PAD - E
    rowf = jnp.concatenate([ei[0], jnp.zeros((pad,), jnp.int32)]).reshape(
        TOT_CHUNKS, PCH)
    colf = jnp.concatenate([ei[1], jnp.full((pad,), N, jnp.int32)]).reshape(
        TOT_CHUNKS, PCH)
    colt = colf.reshape(NW, K, CH)

    zerosD = jnp.zeros((ZR, D), jnp.float32)
    onesD = jnp.ones((CH, D), jnp.float32)
    perm = jnp.asarray(_PERM_NP)

    degp = _sc_deg(colt, zerosD, onesD)
    h1, h1p, dinv = _tc_a(x, W0, b0.reshape(1, D), degp, perm)
    aggp = _sc_prop(_pack_words(h1p), rowf, colf, zerosD)
    hmu, hvar, hmup, hvarp = _tc_b(aggp, h1, dinv, W_mu, b_mu.reshape(1, D),
                                   W_var, b_var.reshape(1, D), perm)
    amup = _sc_prop(_pack_words(hmup), rowf, colf, zerosD)
    avap = _sc_prop(_pack_words(hvarp), rowf, colf, zerosD)
    mu, var = _tc_c(amup, avap, hmu, hvar, dinv)
    return (mu, var)


# speedup vs baseline: 1.0834x; 1.0009x over previous
import jax
import jax.numpy as jnp
import numpy as np
from jax import lax
from jax.experimental import pallas as pl
from jax.experimental.pallas import tpu as pltpu
from jax.experimental.pallas import tpu_sc as plsc

N = 10000
D = 128
E = 320000
HW = D // 2

NC = 2
NS = 16
NW = NC * NS
CH = 128
K = 80
PCH = 64
PK = 160
KP = 40
PHF = PK // KP
TOT_CHUNKS = NW * PK
E_PAD = TOT_CHUNKS * PCH
NBUF = 4

ACC_ROWS = 10240
ZR = ACC_ROWS // NS

_MESH = plsc.VectorSubcoreMesh(core_axis_name="c", subcore_axis_name="s")

_PERM_NP = np.zeros((D, D), np.float32)
for _k in range(D // 32):
    for _i in range(16):
        _PERM_NP[32 * _k + _i, 32 * _k + 2 * _i] = 1.0
        _PERM_NP[32 * _k + 16 + _i, 32 * _k + 2 * _i + 1] = 1.0


def _deg_body(colt_hbm, zeros_hbm, ones_hbm, out_hbm, coli, ones_v, acc, semd):
    c = lax.axis_index("c")
    s = lax.axis_index("s")
    wid = c * NS + s
    pltpu.sync_copy(zeros_hbm, acc.at[pl.ds(s * ZR, ZR)])
    pltpu.sync_copy(colt_hbm.at[wid], coli)
    pltpu.sync_copy(ones_hbm, ones_v)
    plsc.subcore_barrier()

    @pl.loop(0, K)
    def _(j):
        pltpu.async_copy(ones_v, acc.at[coli.at[j]], semd, add=True)

    @pl.loop(0, K)
    def _(j):
        pltpu.make_async_copy(ones_v, acc.at[coli.at[j]], semd).wait()

    plsc.subcore_barrier()
    pltpu.sync_copy(acc.at[pl.ds(s * ZR, ZR)], out_hbm.at[c, pl.ds(s * ZR, ZR)])


_sc_deg = pl.kernel(
    _deg_body,
    out_type=jax.ShapeDtypeStruct((NC, ACC_ROWS, D), jnp.float32),
    mesh=_MESH,
    scratch_types=[
        pltpu.VMEM((K, CH), jnp.int32),
        pltpu.VMEM((CH, D), jnp.float32),
        pltpu.VMEM_SHARED((ACC_ROWS, D), jnp.float32),
        pltpu.SemaphoreType.DMA,
    ],
)


def _prop_body(h_hbm, rowf_hbm, colf_hbm, zeros_hbm, out_hbm,
               rowi, coli, rowsw, stag, acc, semg, sems):
    c = lax.axis_index("c")
    s = lax.axis_index("s")
    pltpu.sync_copy(zeros_hbm, acc.at[pl.ds(s * ZR, ZR)])
    plsc.subcore_barrier()

    wid = c * NS + s
    base = wid * PK

    for ph in range(PHF):
        pltpu.sync_copy(rowf_hbm.at[pl.ds(base + ph * KP, KP)], rowi)
        pltpu.sync_copy(colf_hbm.at[pl.ds(base + ph * KP, KP)], coli)

        for j in range(NBUF - 1):
            pltpu.async_copy(h_hbm.at[rowi.at[j]], rowsw.at[j], semg)

        @pl.loop(0, KP)
        def _(j):
            p = lax.rem(j, NBUF)
            pltpu.make_async_copy(
                h_hbm.at[rowi.at[j]], rowsw.at[p], semg).wait()

            @pl.when(j + NBUF - 1 < KP)
            def _():
                pltpu.async_copy(h_hbm.at[rowi.at[j + NBUF - 1]],
                                 rowsw.at[lax.rem(j + NBUF - 1, NBUF)],
                                 semg)

            q = lax.rem(j, 2)

            @pl.when(j >= 2)
            def _():
                pltpu.make_async_copy(
                    stag.at[q], acc.at[coli.at[j - 2]], sems).wait()

            @pl.loop(0, PCH, unroll=8)
            def _(r):
                for k in range(D // 32):
                    w = rowsw[p, r, pl.ds(16 * k, 16)]
                    a = plsc.bitcast(lax.shift_left(w, 16), jnp.float32)
                    b = plsc.bitcast(
                        lax.bitwise_and(w, jnp.int32(-65536)), jnp.float32)
                    stag[q, r, pl.ds(32 * k, 16)] = a
                    stag[q, r, pl.ds(32 * k + 16, 16)] = b

            pltpu.async_copy(stag.at[q], acc.at[coli.at[j]], sems, add=True)

        for t in (KP - 2, KP - 1):
            pltpu.make_async_copy(
                stag.at[t % 2], acc.at[coli.at[t]], sems).wait()

    plsc.subcore_barrier()
    pltpu.sync_copy(acc.at[pl.ds(s * ZR, ZR)], out_hbm.at[c, pl.ds(s * ZR, ZR)])


_sc_prop = pl.kernel(
    _prop_body,
    out_type=jax.ShapeDtypeStruct((NC, ACC_ROWS, D), jnp.float32),
    mesh=_MESH,
    scratch_types=[
        pltpu.VMEM((KP, PCH), jnp.int32),
        pltpu.VMEM((KP, PCH), jnp.int32),
        pltpu.VMEM((NBUF, PCH, HW), jnp.int32),
        pltpu.VMEM((2, PCH, D), jnp.float32),
        pltpu.VMEM_SHARED((ACC_ROWS, D), jnp.float32),
        pltpu.SemaphoreType.DMA,
        pltpu.SemaphoreType.DMA,
    ],
    compiler_params=pltpu.CompilerParams(use_tc_tiling_on_sc=False,
                                         needs_layout_passes=False),
)

_BLK = 2000
_GRID = N // _BLK
_EPS = 1e-12


def _tc_a_body(x_ref, w_ref, b_ref, degp_ref, perm_ref, h1_ref, h1p_ref,
               dinv_ref):
    xb = x_ref[...]
    nrm = jnp.sqrt(jnp.sum(xb * xb, axis=1, keepdims=True))
    xn = xb / jnp.maximum(nrm, _EPS)
    z = jnp.dot(xn, w_ref[...], preferred_element_type=jnp.float32) + b_ref[...]
    deg = degp_ref[0] + degp_ref[1] + 1.0
    dinv = lax.rsqrt(deg)
    dinv_ref[...] = dinv[:, 0:16]
    h1 = z * dinv
    h1_ref[...] = h1
    h1p_ref[...] = jnp.dot(h1, perm_ref[...],
                           preferred_element_type=jnp.float32).astype(jnp.bfloat16)


_tc_a = pl.pallas_call(
    _tc_a_body,
    grid=(_GRID,),
    in_specs=[
        pl.BlockSpec((_BLK, D), lambda i: (i, 0)),
        pl.BlockSpec((D, D), lambda i: (0, 0)),
        pl.BlockSpec((1, D), lambda i: (0, 0)),
        pl.BlockSpec((NC, _BLK, D), lambda i: (0, i, 0)),
        pl.BlockSpec((D, D), lambda i: (0, 0)),
    ],
    out_specs=[
        pl.BlockSpec((_BLK, D), lambda i: (i, 0)),
        pl.BlockSpec((_BLK, D), lambda i: (i, 0)),
        pl.BlockSpec((_BLK, 16), lambda i: (i, 0)),
    ],
    out_shape=[
        jax.ShapeDtypeStruct((N, D), jnp.float32),
        jax.ShapeDtypeStruct((N, D), jnp.bfloat16),
        jax.ShapeDtypeStruct((N, 16), jnp.float32),
    ],
)


def _tc_b_body(aggp_ref, h1_ref, dinv_ref, wmu_ref, bmu_ref, wvar_ref,
               bvar_ref, perm_ref, hmu_ref, hvar_ref, hmup_ref, hvarp_ref):
    dinv = dinv_ref[...][:, 0:1]
    agg = aggp_ref[0] + aggp_ref[1] + h1_ref[...]
    hid = jnp.maximum(agg * dinv, 0.0)
    perm = perm_ref[...]
    zmu = jnp.dot(hid, wmu_ref[...], preferred_element_type=jnp.float32) + bmu_ref[...]
    nmu = jnp.sqrt(jnp.sum(zmu * zmu, axis=1, keepdims=True))
    hmu = zmu / jnp.maximum(nmu, _EPS) * dinv
    hmu_ref[...] = hmu
    hmup_ref[...] = jnp.dot(hmu, perm,
                            preferred_element_type=jnp.float32).astype(jnp.bfloat16)
    zva = jnp.dot(hid, wvar_ref[...], preferred_element_type=jnp.float32) + bvar_ref[...]
    nva = jnp.sqrt(jnp.sum(zva * zva, axis=1, keepdims=True))
    hvar = zva / jnp.maximum(nva, _EPS) * dinv
    hvar_ref[...] = hvar
    hvarp_ref[...] = jnp.dot(hvar, perm,
                             preferred_element_type=jnp.float32).astype(jnp.bfloat16)


_tc_b = pl.pallas_call(
    _tc_b_body,
    grid=(_GRID,),
    in_specs=[
        pl.BlockSpec((NC, _BLK, D), lambda i: (0, i, 0)),
        pl.BlockSpec((_BLK, D), lambda i: (i, 0)),
        pl.BlockSpec((_BLK, 16), lambda i: (i, 0)),
        pl.BlockSpec((D, D), lambda i: (0, 0)),
        pl.BlockSpec((1, D), lambda i: (0, 0)),
        pl.BlockSpec((D, D), lambda i: (0, 0)),
        pl.BlockSpec((1, D), lambda i: (0, 0)),
        pl.BlockSpec((D, D), lambda i: (0, 0)),
    ],
    out_specs=[
        pl.BlockSpec((_BLK, D), lambda i: (i, 0)),
        pl.BlockSpec((_BLK, D), lambda i: (i, 0)),
        pl.BlockSpec((_BLK, D), lambda i: (i, 0)),
        pl.BlockSpec((_BLK, D), lambda i: (i, 0)),
    ],
    out_shape=[
        jax.ShapeDtypeStruct((N, D), jnp.float32),
        jax.ShapeDtypeStruct((N, D), jnp.float32),
        jax.ShapeDtypeStruct((N, D), jnp.bfloat16),
        jax.ShapeDtypeStruct((N, D), jnp.bfloat16),
    ],
)


def _tc_c_body(amup_ref, avap_ref, hmu_ref, hvar_ref, dinv_ref,
               mu_ref, var_ref):
    dinv = dinv_ref[...][:, 0:1]
    mu_ref[...] = (amup_ref[0] + amup_ref[1] + hmu_ref[...]) * dinv
    var_ref[...] = (avap_ref[0] + avap_ref[1] + hvar_ref[...]) * dinv


_tc_c = pl.pallas_call(
    _tc_c_body,
    grid=(_GRID,),
    in_specs=[
        pl.BlockSpec((NC, _BLK, D), lambda i: (0, i, 0)),
        pl.BlockSpec((NC, _BLK, D), lambda i: (0, i, 0)),
        pl.BlockSpec((_BLK, D), lambda i: (i, 0)),
        pl.BlockSpec((_BLK, D), lambda i: (i, 0)),
        pl.BlockSpec((_BLK, 16), lambda i: (i, 0)),
    ],
    out_specs=[
        pl.BlockSpec((_BLK, D), lambda i: (i, 0)),
        pl.BlockSpec((_BLK, D), lambda i: (i, 0)),
    ],
    out_shape=[
        jax.ShapeDtypeStruct((N, D), jnp.float32),
        jax.ShapeDtypeStruct((N, D), jnp.float32),
    ],
)


def _pack_words(hp):
    return lax.bitcast_convert_type(hp.reshape(N, HW, 2), jnp.int32)


def kernel(x, edge_index, W0, b0, W_mu, b_mu, W_var, b_var):
    ei = edge_index.astype(jnp.int32)
    pad = E_PAD - E
    rowf = jnp.concatenate([ei[0], jnp.zeros((pad,), jnp.int32)]).reshape(
        TOT_CHUNKS, PCH)
    colf = jnp.concatenate([ei[1], jnp.full((pad,), N, jnp.int32)]).reshape(
        TOT_CHUNKS, PCH)
    colt = colf.reshape(NW, K, CH)

    zerosD = jnp.zeros((ZR, D), jnp.float32)
    onesD = jnp.ones((CH, D), jnp.float32)
    perm = jnp.asarray(_PERM_NP)

    degp = _sc_deg(colt, zerosD, onesD)
    h1, h1p, dinv = _tc_a(x, W0, b0.reshape(1, D), degp, perm)
    aggp = _sc_prop(_pack_words(h1p), rowf, colf, zerosD)
    hmu, hvar, hmup, hvarp = _tc_b(aggp, h1, dinv, W_mu, b_mu.reshape(1, D),
                                   W_var, b_var.reshape(1, D), perm)
    amup = _sc_prop(_pack_words(hmup), rowf, colf, zerosD)
    avap = _sc_prop(_pack_words(hvarp), rowf, colf, zerosD)
    mu, var = _tc_c(amup, avap, hmu, hvar, dinv)
    return (mu, var)
